# Initial kernel scaffold; baseline (speedup 1.0000x reference)
import jax
import jax.numpy as jnp
from jax.experimental import pallas as pl


def kernel(queries, keys):
    raise NotImplementedError("write your pallas kernel here")


# speedup vs baseline: 5.8661x; 5.8661x over previous
import functools

import jax
import jax.numpy as jnp
from jax.experimental import pallas as pl
from jax.experimental.pallas import tpu as pltpu

_BLOCK_K = 2048


def _nn_body(n_k, q_ref, kt_ref, scores_ref, idx_ref, runmin_ref, runidx_ref):
    i = pl.program_id(0)
    nblocks = pl.num_programs(0)

    q = q_ref[...]
    kt = kt_ref[...]
    col16 = jax.lax.broadcasted_iota(jnp.int32, kt.shape, 1) + i * _BLOCK_K
    kt = jnp.where(col16 < n_k, kt, 100.0)

    qk = jax.lax.dot_general(
        q, kt, (((1,), (0,)), ((), ())),
        preferred_element_type=jnp.float32)
    q_sq = jnp.sum(q * q, axis=1, keepdims=True)
    k_sq = jnp.sum(kt * kt, axis=0, keepdims=True)
    d = (q_sq - 2.0 * qk) + k_sq

    m = jnp.min(d, axis=1, keepdims=True)
    col = jax.lax.broadcasted_iota(jnp.int32, d.shape, 1)
    am = jnp.min(jnp.where(d == m, col, jnp.int32(2**30)),
                 axis=1, keepdims=True)
    am = am + i * _BLOCK_K

    @pl.when(i == 0)
    def _():
        runmin_ref[...] = m
        runidx_ref[...] = am

    @pl.when(i > 0)
    def _():
        better = m < runmin_ref[...]
        runidx_ref[...] = jnp.where(better, am, runidx_ref[...])
        runmin_ref[...] = jnp.where(better, m, runmin_ref[...])

    @pl.when(i == nblocks - 1)
    def _():
        best = jnp.maximum(runmin_ref[...], 0.0)
        scores_ref[...] = jnp.sqrt(best + 1e-12)[:, 0]
        idx_ref[...] = runidx_ref[...]


def kernel(queries, keys):
    n_q, dim = queries.shape
    n_k = keys.shape[0]
    kt = keys.T

    grid = pl.cdiv(n_k, _BLOCK_K)
    scores, idx = pl.pallas_call(
        functools.partial(_nn_body, n_k),
        grid=(grid,),
        in_specs=[
            pl.BlockSpec((n_q, dim), lambda i: (0, 0)),
            pl.BlockSpec((dim, _BLOCK_K), lambda i: (0, i)),
        ],
        out_specs=[
            pl.BlockSpec((n_q,), lambda i: (0,)),
            pl.BlockSpec((n_q, 1), lambda i: (0, 0)),
        ],
        out_shape=[
            jax.ShapeDtypeStruct((n_q,), jnp.float32),
            jax.ShapeDtypeStruct((n_q, 1), jnp.int32),
        ],
        scratch_shapes=[
            pltpu.VMEM((n_q, 1), jnp.float32),
            pltpu.VMEM((n_q, 1), jnp.int32),
        ],
        compiler_params=pltpu.CompilerParams(
            dimension_semantics=("arbitrary",),
        ),
    )(queries, kt)
    return (scores, idx)


# speedup vs baseline: 7.0451x; 1.2010x over previous
import functools

import jax
import jax.numpy as jnp
from jax.experimental import pallas as pl
from jax.experimental.pallas import tpu as pltpu

_BLOCK_K = 2048


def _nn_body(n_k, q_ref, kt_ref, min_ref, idx_ref):
    i = pl.program_id(0)

    q = q_ref[...]
    kt = kt_ref[...]
    col16 = jax.lax.broadcasted_iota(jnp.int32, kt.shape, 1) + i * _BLOCK_K
    kt = jnp.where(col16 < n_k, kt, 100.0)

    qk = jax.lax.dot_general(
        q, kt, (((1,), (0,)), ((), ())),
        preferred_element_type=jnp.float32)
    q_sq = jnp.sum(q * q, axis=1, keepdims=True)
    k_sq = jnp.sum(kt * kt, axis=0, keepdims=True)
    d = (q_sq - 2.0 * qk) + k_sq

    m = jnp.min(d, axis=1, keepdims=True)
    coli = (jax.lax.broadcasted_iota(jnp.int32, (1, kt.shape[1]), 1)
            + i * _BLOCK_K)
    colf = coli.astype(jnp.float32)
    am = jnp.min(jnp.where(d == m, colf, 3e9), axis=1, keepdims=True)

    @pl.when(i == 0)
    def _():
        min_ref[...] = m
        idx_ref[...] = am

    @pl.when(i > 0)
    def _():
        better = m < min_ref[...]
        idx_ref[...] = jnp.where(better, am, idx_ref[...])
        min_ref[...] = jnp.where(better, m, min_ref[...])


def kernel(queries, keys):
    n_q, dim = queries.shape
    n_k = keys.shape[0]
    kt = keys.T

    grid = pl.cdiv(n_k, _BLOCK_K)
    minv, idxf = pl.pallas_call(
        functools.partial(_nn_body, n_k),
        grid=(grid,),
        in_specs=[
            pl.BlockSpec((n_q, dim), lambda i: (0, 0)),
            pl.BlockSpec((dim, _BLOCK_K), lambda i: (0, i)),
        ],
        out_specs=[
            pl.BlockSpec((n_q, 1), lambda i: (0, 0)),
            pl.BlockSpec((n_q, 1), lambda i: (0, 0)),
        ],
        out_shape=[
            jax.ShapeDtypeStruct((n_q, 1), jnp.float32),
            jax.ShapeDtypeStruct((n_q, 1), jnp.float32),
        ],
        compiler_params=pltpu.CompilerParams(
            dimension_semantics=("arbitrary",),
        ),
    )(queries, kt)
    scores = jnp.sqrt(jnp.maximum(minv[:, 0], 0.0) + 1e-12)
    idx = idxf.astype(jnp.int32)
    return (scores, idx)


# speedup vs baseline: 9.5411x; 1.3543x over previous
import functools

import jax
import jax.numpy as jnp
from jax.experimental import pallas as pl
from jax.experimental.pallas import tpu as pltpu

_BLOCK_K = 8192


def _nn_body(n_k, q_ref, kt_ref, min_ref, idx_ref):
    i = pl.program_id(0)

    q = q_ref[...]
    kt2 = kt_ref[...]
    col16 = jax.lax.broadcasted_iota(jnp.int32, kt2.shape, 1) + i * _BLOCK_K
    kt2 = jnp.where(col16 < n_k, kt2, -200.0)

    qk2 = jax.lax.dot_general(
        q, kt2, (((1,), (0,)), ((), ())),
        preferred_element_type=jnp.float32)
    q_sq = jnp.sum(q * q, axis=1, keepdims=True)
    k_sq = 0.25 * jnp.sum(kt2 * kt2, axis=0, keepdims=True)

    chunk = 128
    nch = _BLOCK_K // chunk
    lanef = jax.lax.broadcasted_iota(
        jnp.int32, (1, chunk), 1).astype(jnp.float32)
    basef = (i * _BLOCK_K).astype(jnp.float32)
    accv = acci = None
    for c in range(nch):
        qkc = qk2[:, c * chunk:(c + 1) * chunk]
        ksqc = k_sq[:, c * chunk:(c + 1) * chunk]
        dch = (q_sq + qkc) + ksqc
        idx_row = lanef + (basef + float(c * chunk))
        if c == 0:
            accv = dch
            acci = jnp.broadcast_to(idx_row, dch.shape)
        else:
            mask = dch < accv
            acci = jnp.where(mask, idx_row, acci)
            accv = jnp.minimum(accv, dch)
    m = jnp.min(accv, axis=1, keepdims=True)
    am = jnp.min(jnp.where(accv == m, acci, 3e9),
                 axis=1, keepdims=True)

    @pl.when(i == 0)
    def _():
        min_ref[...] = m
        idx_ref[...] = am

    @pl.when(i > 0)
    def _():
        better = m < min_ref[...]
        idx_ref[...] = jnp.where(better, am, idx_ref[...])
        min_ref[...] = jnp.where(better, m, min_ref[...])


def kernel(queries, keys):
    n_q, dim = queries.shape
    n_k = keys.shape[0]
    kt = -2.0 * keys.T

    grid = pl.cdiv(n_k, _BLOCK_K)
    minv, idxf = pl.pallas_call(
        functools.partial(_nn_body, n_k),
        grid=(grid,),
        in_specs=[
            pl.BlockSpec((n_q, dim), lambda i: (0, 0)),
            pl.BlockSpec((dim, _BLOCK_K), lambda i: (0, i)),
        ],
        out_specs=[
            pl.BlockSpec((n_q, 1), lambda i: (0, 0)),
            pl.BlockSpec((n_q, 1), lambda i: (0, 0)),
        ],
        out_shape=[
            jax.ShapeDtypeStruct((n_q, 1), jnp.float32),
            jax.ShapeDtypeStruct((n_q, 1), jnp.float32),
        ],
        compiler_params=pltpu.CompilerParams(
            dimension_semantics=("arbitrary",),
        ),
    )(queries, kt)
    scores = jnp.sqrt(jnp.maximum(minv[:, 0], 0.0) + 1e-12)
    idx = idxf.astype(jnp.int32)
    return (scores, idx)


# speedup vs baseline: 9.7015x; 1.0168x over previous
import functools

import jax
import jax.numpy as jnp
from jax.experimental import pallas as pl
from jax.experimental.pallas import tpu as pltpu

_BLOCK_K = 10240


def _nn_body(n_k, q_ref, kt_ref, min_ref, idx_ref):
    i = pl.program_id(0)

    q = q_ref[...]
    kt2 = kt_ref[...]
    col16 = jax.lax.broadcasted_iota(jnp.int32, kt2.shape, 1) + i * _BLOCK_K
    kt2 = jnp.where(col16 < n_k, kt2, -200.0)

    qk2 = jax.lax.dot_general(
        q, kt2, (((1,), (0,)), ((), ())),
        preferred_element_type=jnp.float32)
    q_sq = jnp.sum(q * q, axis=1, keepdims=True)
    k_sq = 0.25 * jnp.sum(kt2 * kt2, axis=0, keepdims=True)

    chunk = 128
    nch = _BLOCK_K // chunk
    lanef = jax.lax.broadcasted_iota(
        jnp.int32, (1, chunk), 1).astype(jnp.float32)
    basef = (i * _BLOCK_K).astype(jnp.float32)
    accv = acci = None
    for c in range(nch):
        qkc = qk2[:, c * chunk:(c + 1) * chunk]
        ksqc = k_sq[:, c * chunk:(c + 1) * chunk]
        dch = (q_sq + qkc) + ksqc
        idx_row = lanef + (basef + float(c * chunk))
        if c == 0:
            accv = dch
            acci = jnp.broadcast_to(idx_row, dch.shape)
        else:
            mask = dch < accv
            acci = jnp.where(mask, idx_row, acci)
            accv = jnp.minimum(accv, dch)
    m = jnp.min(accv, axis=1, keepdims=True)
    am = jnp.min(jnp.where(accv == m, acci, 3e9),
                 axis=1, keepdims=True)

    @pl.when(i == 0)
    def _():
        min_ref[...] = m
        idx_ref[...] = am

    @pl.when(i > 0)
    def _():
        better = m < min_ref[...]
        idx_ref[...] = jnp.where(better, am, idx_ref[...])
        min_ref[...] = jnp.where(better, m, min_ref[...])


def kernel(queries, keys):
    n_q, dim = queries.shape
    n_k = keys.shape[0]
    kt = -2.0 * keys.T

    grid = pl.cdiv(n_k, _BLOCK_K)
    minv, idxf = pl.pallas_call(
        functools.partial(_nn_body, n_k),
        grid=(grid,),
        in_specs=[
            pl.BlockSpec((n_q, dim), lambda i: (0, 0)),
            pl.BlockSpec((dim, _BLOCK_K), lambda i: (0, i)),
        ],
        out_specs=[
            pl.BlockSpec((n_q, 1), lambda i: (0, 0)),
            pl.BlockSpec((n_q, 1), lambda i: (0, 0)),
        ],
        out_shape=[
            jax.ShapeDtypeStruct((n_q, 1), jnp.float32),
            jax.ShapeDtypeStruct((n_q, 1), jnp.float32),
        ],
        compiler_params=pltpu.CompilerParams(
            dimension_semantics=("arbitrary",),
        ),
    )(queries, kt)
    scores = jnp.sqrt(jnp.maximum(minv[:, 0], 0.0) + 1e-12)
    idx = idxf.astype(jnp.int32)
    return (scores, idx)
